# baseline (speedup 1.0000x reference)
import jax
import jax.numpy as jnp
from jax import lax
from jax.experimental import pallas as pl
from jax.experimental.pallas import tpu as pltpu
from jax.experimental.pallas import tpu_sc as plsc

NUM_EMB = 1000000
DIM = 32

NC = 2
NS = 16
NW = NC * NS

B_TOTAL = 4096 * 50
B_PER_W = B_TOTAL // NW
CHUNK = 128
N_CHUNK = B_PER_W // CHUNK
GROUP = 10
N_GROUP = N_CHUNK // GROUP


def _body(x_hbm, w_hbm, out_hbm, idx_v, rows_v, sem):
    c = lax.axis_index("c")
    s = lax.axis_index("s")
    wid = s * NC + c

    pltpu.sync_copy(x_hbm.at[wid], idx_v)

    for g in range(N_GROUP):
        cps = []
        for j in range(GROUP):
            cp = pltpu.async_copy(
                w_hbm.at[idx_v.at[g * GROUP + j]],
                rows_v.at[pl.ds(j * CHUNK, CHUNK)],
                sem,
            )
            cps.append(cp)
        for cp in cps:
            cp.wait()
        pltpu.sync_copy(
            rows_v,
            out_hbm.at[pl.ds(wid * B_PER_W + g * GROUP * CHUNK, GROUP * CHUNK)],
        )


@jax.jit
def _run(x_r, weight):
    mesh = plsc.VectorSubcoreMesh(core_axis_name="c", subcore_axis_name="s")
    return pl.kernel(
        _body,
        out_type=jax.ShapeDtypeStruct((B_TOTAL, DIM), jnp.float32),
        mesh=mesh,
        compiler_params=pltpu.CompilerParams(use_tc_tiling_on_sc=False),
        scratch_types=[
            pltpu.VMEM((N_CHUNK, CHUNK), jnp.int32),
            pltpu.VMEM((GROUP * CHUNK, DIM), jnp.float32),
            pltpu.SemaphoreType.DMA,
        ],
    )(x_r, weight)


def kernel(x, weight):
    B, S = x.shape
    x_r = x.astype(jnp.int32).reshape(NW, N_CHUNK, CHUNK)
    out = _run(x_r, weight)
    return out.reshape(B, S, DIM)
